# baseline (speedup 1.0000x reference)
import functools

import jax
import jax.numpy as jnp
from jax.experimental import pallas as pl
from jax.experimental.pallas import tpu as pltpu

_TN = 1600


def _tbcnn_block(nodes_ref, mb_ref, wl_ref, wr_ref, wt_ref, b_ref, out_ref,
                 *, c):
    mb = mb_ref[...]
    coef = (jax.lax.broadcasted_iota(jnp.int32, (1, c, 1), 1)
            .astype(jnp.float32)) / (c - 1)
    s = jnp.sum(mb, axis=1)
    a = jnp.sum(mb * coef, axis=1)
    wl = wl_ref[...]
    acc = jnp.dot(s, wl, preferred_element_type=jnp.float32)
    acc += jnp.dot(a, wr_ref[...] - wl, preferred_element_type=jnp.float32)
    acc += jnp.dot(nodes_ref[...], wt_ref[...], preferred_element_type=jnp.float32)
    out_ref[...] = jnp.maximum(acc + b_ref[...], 0.0)


def kernel(nodes_h, mailbox_h, W_left, W_right, W_top, b_conv):
    n, c, h = mailbox_h.shape
    return pl.pallas_call(
        functools.partial(_tbcnn_block, c=c),
        grid=(pl.cdiv(n, _TN),),
        in_specs=[
            pl.BlockSpec((_TN, h), lambda i: (i, 0)),
            pl.BlockSpec((_TN, c, h), lambda i: (i, 0, 0)),
            pl.BlockSpec((h, h), lambda i: (0, 0)),
            pl.BlockSpec((h, h), lambda i: (0, 0)),
            pl.BlockSpec((h, h), lambda i: (0, 0)),
            pl.BlockSpec((1, h), lambda i: (0, 0)),
        ],
        out_specs=pl.BlockSpec((_TN, h), lambda i: (i, 0)),
        out_shape=jax.ShapeDtypeStruct((n, h), jnp.float32),
        compiler_params=pltpu.CompilerParams(
            dimension_semantics=("parallel",),
        ),
    )(nodes_h, mailbox_h, W_left, W_right, W_top, b_conv)


# speedup vs baseline: 1.0996x; 1.0996x over previous
import functools

import jax
import jax.numpy as jnp
from jax.experimental import pallas as pl
from jax.experimental.pallas import tpu as pltpu

_TN = 1000


def _tbcnn_block(nodes_ref, mb_ref, wl_ref, wr_ref, wt_ref, b_ref, out_ref,
                 *, c):
    mb = mb_ref[...]
    coef = (jax.lax.broadcasted_iota(jnp.int32, (1, c, 1), 1)
            .astype(jnp.float32)) / (c - 1)
    s = jnp.sum(mb, axis=1)
    a = jnp.sum(mb * coef, axis=1)
    wl = wl_ref[...]
    acc = jnp.dot(s, wl, preferred_element_type=jnp.float32)
    acc += jnp.dot(a, wr_ref[...] - wl, preferred_element_type=jnp.float32)
    acc += jnp.dot(nodes_ref[...], wt_ref[...], preferred_element_type=jnp.float32)
    out_ref[...] = jnp.maximum(acc + b_ref[...], 0.0)


def kernel(nodes_h, mailbox_h, W_left, W_right, W_top, b_conv):
    n, c, h = mailbox_h.shape
    return pl.pallas_call(
        functools.partial(_tbcnn_block, c=c),
        grid=(pl.cdiv(n, _TN),),
        in_specs=[
            pl.BlockSpec((_TN, h), lambda i: (i, 0)),
            pl.BlockSpec((_TN, c, h), lambda i: (i, 0, 0)),
            pl.BlockSpec((h, h), lambda i: (0, 0)),
            pl.BlockSpec((h, h), lambda i: (0, 0)),
            pl.BlockSpec((h, h), lambda i: (0, 0)),
            pl.BlockSpec((1, h), lambda i: (0, 0)),
        ],
        out_specs=pl.BlockSpec((_TN, h), lambda i: (i, 0)),
        out_shape=jax.ShapeDtypeStruct((n, h), jnp.float32),
        compiler_params=pltpu.CompilerParams(
            dimension_semantics=("parallel",),
        ),
    )(nodes_h, mailbox_h, W_left, W_right, W_top, b_conv)
